# baseline (speedup 1.0000x reference)
import functools

import jax
import jax.numpy as jnp
from jax import lax
from jax.experimental import pallas as pl
from jax.experimental.pallas import tpu as pltpu
from jax.experimental.pallas import tpu_sc as plsc

N = 10000
E = 320000
IN_C = 128
H = 64

NC = 2
NS = 16
NW = NC * NS

NP = 10240
RPT = NP // NS
EPW = E // NW
KR = 125
K = 128
NCK = EPW // KR
G = E // KR
RING = 8
STEADY = NCK // RING - 1

BT = 1024
GRID = NP // BT

_mesh = plsc.VectorSubcoreMesh(core_axis_name="c", subcore_axis_name="s")
_sc_params = pltpu.CompilerParams(
    needs_layout_passes=False, use_tc_tiling_on_sc=False
)


@functools.partial(
    pl.kernel,
    out_type=jax.ShapeDtypeStruct((NC, NP), jnp.float32),
    mesh=_mesh,
    scratch_types=[
        pltpu.VMEM((EPW,), jnp.int32),
        pltpu.VMEM((NP,), jnp.float32),
        pltpu.VMEM((NS, RPT), jnp.float32),
        pltpu.VMEM_SHARED((NS, NP), jnp.float32),
    ],
    compiler_params=_sc_params,
)
def _deg_kernel(dst_hbm, out_hbm, dstv, degv, redv, shared):
    c = lax.axis_index("c")
    s = lax.axis_index("s")
    wid = c * NS + s
    pltpu.sync_copy(dst_hbm.at[pl.ds(wid * EPW, EPW)], dstv)

    zero16 = jnp.zeros((16,), jnp.float32)
    ones16 = jnp.ones((16,), jnp.float32)

    def zbody(i, carry):
        degv[pl.ds(i * 16, 16)] = zero16
        return carry

    lax.fori_loop(0, NP // 16, zbody, 0)

    def hbody(i, carry):
        idx = dstv[pl.ds(i * 16, 16)]
        plsc.addupdate_scatter(degv, [idx], ones16)
        return carry

    lax.fori_loop(0, EPW // 16, hbody, 0)

    pltpu.sync_copy(degv, shared.at[s])
    plsc.subcore_barrier()
    pltpu.sync_copy(shared.at[:, pl.ds(s * RPT, RPT)], redv)

    def rbody(v, carry):
        acc = redv[0, pl.ds(v * 16, 16)]
        for r in range(1, NS):
            acc = acc + redv[r, pl.ds(v * 16, 16)]
        degv[pl.ds(v * 16, 16)] = acc
        return carry

    lax.fori_loop(0, RPT // 16, rbody, 0)
    pltpu.sync_copy(degv.at[pl.ds(0, RPT)], out_hbm.at[c, pl.ds(s * RPT, RPT)])


@functools.partial(
    pl.kernel,
    out_type=jax.ShapeDtypeStruct((NC, NP, H), jnp.float32),
    mesh=_mesh,
    scratch_types=(
        [pltpu.VMEM((2, K), jnp.int32) for _ in range(RING)]
        + [pltpu.VMEM((K, H), jnp.float32) for _ in range(RING)]
        + [pltpu.VMEM_SHARED((NP, H), jnp.float32)]
        + [pltpu.SemaphoreType.DMA for _ in range(2 * RING)]
    ),
    compiler_params=_sc_params,
)
def _edge_kernel(p_hbm, ei_hbm, z_hbm, out_hbm, *refs):
    ibufs = refs[0:RING]
    rbufs = refs[RING:2 * RING]
    acc = refs[2 * RING]
    isems = refs[2 * RING + 1:2 * RING + 1 + RING]
    gsems = refs[2 * RING + 1 + RING:2 * RING + 1 + 2 * RING]

    c = lax.axis_index("c")
    s = lax.axis_index("s")
    wid = c * NS + s
    g0 = wid * NCK
    rslice = pl.ds(s * RPT, RPT)

    def issue_idx(g, m):
        pltpu.async_copy(ei_hbm.at[g], ibufs[m], isems[m])

    def wait_idx(m):
        pltpu.make_async_copy(ei_hbm.at[0], ibufs[m], isems[m]).wait()

    def issue_gather(m, b):
        pltpu.async_copy(p_hbm.at[ibufs[m].at[0]], rbufs[b], gsems[b])

    def wait_gather(b):
        pltpu.make_async_copy(p_hbm.at[pl.ds(0, K)], rbufs[b], gsems[b]).wait()

    def scatter_add(b, m):
        pltpu.sync_copy(rbufs[b], acc.at[ibufs[m].at[1]], add=True)

    for m in range(6):
        issue_idx(g0 + m, m)

    @pl.when(c == 0)
    def _():
        pltpu.sync_copy(p_hbm.at[rslice], acc.at[rslice])

    @pl.when(c != 0)
    def _():
        pltpu.sync_copy(z_hbm.at[rslice], acc.at[rslice])

    for m in range(5):
        wait_idx(m)
        issue_gather(m, m)

    plsc.subcore_barrier()

    def outer(t, carry):
        jb = t * RING
        for u in range(RING):
            m = u
            mp5 = (u + 5) % RING
            mp6 = (u + 6) % RING
            wait_idx(mp5)
            issue_gather(mp5, mp5)
            issue_idx(g0 + jb + u + 6, mp6)
            wait_gather(m)
            scatter_add(m, m)
        return carry

    lax.fori_loop(0, STEADY, outer, 0)

    for j in range(NCK - RING, NCK):
        m = j % RING
        if j + 5 < NCK:
            mp5 = (j + 5) % RING
            wait_idx(mp5)
            issue_gather(mp5, mp5)
        if j + 6 < NCK:
            issue_idx(g0 + j + 6, (j + 6) % RING)
        wait_gather(m)
        scatter_add(m, m)

    plsc.subcore_barrier()
    pltpu.sync_copy(acc.at[rslice], out_hbm.at[c, rslice])


def _dis_block(d_ref):
    return lax.rsqrt(d_ref[0, :] + d_ref[1, :] + 1.0)


def _tc_first_body(x_ref, w_ref, d_ref, o_ref):
    dis = _dis_block(d_ref)
    h = jnp.dot(x_ref[...], w_ref[...], preferred_element_type=jnp.float32)
    o_ref[...] = h * dis[:, None]


def _tc_first(xp, w1, deg2):
    return pl.pallas_call(
        _tc_first_body,
        grid=(GRID,),
        in_specs=[
            pl.BlockSpec((BT, IN_C), lambda i: (i, 0)),
            pl.BlockSpec((IN_C, H), lambda i: (0, 0)),
            pl.BlockSpec((NC, BT), lambda i: (0, i)),
        ],
        out_specs=pl.BlockSpec((BT, H), lambda i: (i, 0)),
        out_shape=jax.ShapeDtypeStruct((NP, H), jnp.float32),
    )(xp, w1, deg2)


def _tc_mid_body(a_ref, d_ref, b_ref, w_ref, o_ref):
    dis = _dis_block(d_ref)
    h = (a_ref[0] + a_ref[1]) * dis[:, None] + b_ref[0, :][None, :]
    act = jnp.maximum(h, 0.0)
    o_ref[...] = jnp.dot(act, w_ref[...],
                         preferred_element_type=jnp.float32) * dis[:, None]


def _tc_mid(acc, deg2, b_prev, w_next):
    return pl.pallas_call(
        _tc_mid_body,
        grid=(GRID,),
        in_specs=[
            pl.BlockSpec((NC, BT, H), lambda i: (0, i, 0)),
            pl.BlockSpec((NC, BT), lambda i: (0, i)),
            pl.BlockSpec((1, H), lambda i: (0, 0)),
            pl.BlockSpec((H, H), lambda i: (0, 0)),
        ],
        out_specs=pl.BlockSpec((BT, H), lambda i: (i, 0)),
        out_shape=jax.ShapeDtypeStruct((NP, H), jnp.float32),
    )(acc, deg2, b_prev, w_next)


def _tc_head_body(a_ref, d_ref, b_ref, wh_ref, bh_ref, s_ref, w_ref):
    dis = _dis_block(d_ref)
    h = (a_ref[0] + a_ref[1]) * dis[:, None] + b_ref[0, :][None, :]
    act = jnp.maximum(h, 0.0)
    o = jnp.dot(act, wh_ref[...],
                preferred_element_type=jnp.float32) + bh_ref[0, :][None, :]
    s_ref[...] = o[:, 0:2]
    w_ref[...] = jax.nn.softplus(o[:, 2:3])


def _tc_head(acc, deg2, b3, wh, bh):
    return pl.pallas_call(
        _tc_head_body,
        grid=(GRID,),
        in_specs=[
            pl.BlockSpec((NC, BT, H), lambda i: (0, i, 0)),
            pl.BlockSpec((NC, BT), lambda i: (0, i)),
            pl.BlockSpec((1, H), lambda i: (0, 0)),
            pl.BlockSpec((H, 3), lambda i: (0, 0)),
            pl.BlockSpec((1, 3), lambda i: (0, 0)),
        ],
        out_specs=[
            pl.BlockSpec((BT, 2), lambda i: (i, 0)),
            pl.BlockSpec((BT, 1), lambda i: (i, 0)),
        ],
        out_shape=[
            jax.ShapeDtypeStruct((NP, 2), jnp.float32),
            jax.ShapeDtypeStruct((NP, 1), jnp.float32),
        ],
    )(acc, deg2, b3, wh, bh)


def kernel(x, edge_index, W1, b1, W2, b2, W3, b3, Ws, bs, Ww, bw):
    xp = jnp.zeros((NP, IN_C), jnp.float32).at[:N].set(x)
    ei = edge_index.astype(jnp.int32)
    pad_row = (N + (jnp.arange(G, dtype=jnp.int32) % (NP - N)))[:, None]
    srcs = jnp.concatenate(
        [ei[0].reshape(G, KR),
         jnp.broadcast_to(pad_row, (G, K - KR))], axis=1)
    dsts = jnp.concatenate(
        [ei[1].reshape(G, KR),
         jnp.broadcast_to(pad_row, (G, K - KR))], axis=1)
    ei_packed = jnp.stack([srcs, dsts], axis=1)
    dst_flat = ei[1]
    zeros_nph = jnp.zeros((NP, H), jnp.float32)
    wh = jnp.concatenate([Ws, Ww], axis=1)
    bh = jnp.concatenate([bs, bw]).reshape(1, 3)
    b1r = b1.reshape(1, H)
    b2r = b2.reshape(1, H)
    b3r = b3.reshape(1, H)

    deg2 = _deg_kernel(dst_flat)
    p1 = _tc_first(xp, W1, deg2)
    acc1 = _edge_kernel(p1, ei_packed, zeros_nph)
    p2 = _tc_mid(acc1, deg2, b1r, W2)
    acc2 = _edge_kernel(p2, ei_packed, zeros_nph)
    p3 = _tc_mid(acc2, deg2, b2r, W3)
    acc3 = _edge_kernel(p3, ei_packed, zeros_nph)
    shifts, weights = _tc_head(acc3, deg2, b3r, wh, bh)
    return (shifts[:N], weights[:N, 0])


# speedup vs baseline: 1.0011x; 1.0011x over previous
import functools

import jax
import jax.numpy as jnp
from jax import lax
from jax.experimental import pallas as pl
from jax.experimental.pallas import tpu as pltpu
from jax.experimental.pallas import tpu_sc as plsc

N = 10000
E = 320000
IN_C = 128
H = 64

NC = 2
NS = 16
NW = NC * NS

NP = 10240
RPT = NP // NS
EPW = E // NW
KR = 125
K = 128
NCK = EPW // KR
G = E // KR
RING = 8
STEADY = NCK // RING - 1

BT = 1024
GRID = NP // BT

_mesh = plsc.VectorSubcoreMesh(core_axis_name="c", subcore_axis_name="s")
_sc_params = pltpu.CompilerParams(
    needs_layout_passes=False, use_tc_tiling_on_sc=False
)


@functools.partial(
    pl.kernel,
    out_type=jax.ShapeDtypeStruct((NC, NP), jnp.float32),
    mesh=_mesh,
    scratch_types=[
        pltpu.VMEM((EPW,), jnp.int32),
        pltpu.VMEM((NP,), jnp.float32),
        pltpu.VMEM((NS, RPT), jnp.float32),
        pltpu.VMEM_SHARED((NS, NP), jnp.float32),
    ],
    compiler_params=_sc_params,
)
def _deg_kernel(dst_hbm, out_hbm, dstv, degv, redv, shared):
    c = lax.axis_index("c")
    s = lax.axis_index("s")
    wid = c * NS + s
    pltpu.sync_copy(dst_hbm.at[pl.ds(wid * EPW, EPW)], dstv)

    zero16 = jnp.zeros((16,), jnp.float32)
    ones16 = jnp.ones((16,), jnp.float32)

    def zbody(i, carry):
        degv[pl.ds(i * 16, 16)] = zero16
        return carry

    lax.fori_loop(0, NP // 16, zbody, 0)

    def hbody(i, carry):
        idx = dstv[pl.ds(i * 16, 16)]
        plsc.addupdate_scatter(degv, [idx], ones16)
        return carry

    lax.fori_loop(0, EPW // 16, hbody, 0)

    pltpu.sync_copy(degv, shared.at[s])
    plsc.subcore_barrier()
    pltpu.sync_copy(shared.at[:, pl.ds(s * RPT, RPT)], redv)

    def rbody(v, carry):
        acc = redv[0, pl.ds(v * 16, 16)]
        for r in range(1, NS):
            acc = acc + redv[r, pl.ds(v * 16, 16)]
        degv[pl.ds(v * 16, 16)] = acc
        return carry

    lax.fori_loop(0, RPT // 16, rbody, 0)
    pltpu.sync_copy(degv.at[pl.ds(0, RPT)], out_hbm.at[c, pl.ds(s * RPT, RPT)])


@functools.partial(
    pl.kernel,
    out_type=jax.ShapeDtypeStruct((NC, NP, H), jnp.float32),
    mesh=_mesh,
    scratch_types=(
        [pltpu.VMEM((2, K), jnp.int32) for _ in range(RING)]
        + [pltpu.VMEM((K, H), jnp.float32) for _ in range(RING)]
        + [pltpu.VMEM_SHARED((NP, H), jnp.float32)]
        + [pltpu.SemaphoreType.DMA for _ in range(2 * RING)]
    ),
    compiler_params=_sc_params,
)
def _edge_kernel(p_hbm, ei_hbm, z_hbm, out_hbm, *refs):
    ibufs = refs[0:RING]
    rbufs = refs[RING:2 * RING]
    acc = refs[2 * RING]
    isems = refs[2 * RING + 1:2 * RING + 1 + RING]
    gsems = refs[2 * RING + 1 + RING:2 * RING + 1 + 2 * RING]

    c = lax.axis_index("c")
    s = lax.axis_index("s")
    wid = c * NS + s
    g0 = wid * NCK
    rslice = pl.ds(s * RPT, RPT)

    def issue_idx(g, m):
        pltpu.async_copy(ei_hbm.at[g], ibufs[m], isems[m])

    def wait_idx(m):
        pltpu.make_async_copy(ei_hbm.at[0], ibufs[m], isems[m]).wait()

    def issue_gather(m, b):
        pltpu.async_copy(p_hbm.at[ibufs[m].at[0]], rbufs[b], gsems[b])

    def wait_gather(b):
        pltpu.make_async_copy(p_hbm.at[pl.ds(0, K)], rbufs[b], gsems[b]).wait()

    def scatter_add(b, m):
        pltpu.sync_copy(rbufs[b], acc.at[ibufs[m].at[1]], add=True)

    for m in range(5):
        issue_idx(g0 + m, m)

    @pl.when(c == 0)
    def _():
        pltpu.sync_copy(p_hbm.at[rslice], acc.at[rslice])

    @pl.when(c != 0)
    def _():
        pltpu.sync_copy(z_hbm.at[rslice], acc.at[rslice])

    for m in range(4):
        wait_idx(m)
        issue_gather(m, m)

    plsc.subcore_barrier()

    def outer(t, carry):
        jb = t * RING
        for u in range(RING):
            m = u
            mp4 = (u + 4) % RING
            mp5 = (u + 5) % RING
            wait_idx(mp4)
            issue_gather(mp4, mp4)
            issue_idx(g0 + jb + u + 5, mp5)
            wait_gather(m)
            scatter_add(m, m)
        return carry

    lax.fori_loop(0, STEADY, outer, 0)

    for j in range(NCK - RING, NCK):
        m = j % RING
        if j + 4 < NCK:
            mp4 = (j + 4) % RING
            wait_idx(mp4)
            issue_gather(mp4, mp4)
        if j + 5 < NCK:
            issue_idx(g0 + j + 5, (j + 5) % RING)
        wait_gather(m)
        scatter_add(m, m)

    plsc.subcore_barrier()
    pltpu.sync_copy(acc.at[rslice], out_hbm.at[c, rslice])


def _dis_block(d_ref):
    return lax.rsqrt(d_ref[0, :] + d_ref[1, :] + 1.0)


def _tc_first_body(x_ref, w_ref, d_ref, o_ref):
    dis = _dis_block(d_ref)
    h = jnp.dot(x_ref[...], w_ref[...], preferred_element_type=jnp.float32)
    o_ref[...] = h * dis[:, None]


def _tc_first(xp, w1, deg2):
    return pl.pallas_call(
        _tc_first_body,
        grid=(GRID,),
        in_specs=[
            pl.BlockSpec((BT, IN_C), lambda i: (i, 0)),
            pl.BlockSpec((IN_C, H), lambda i: (0, 0)),
            pl.BlockSpec((NC, BT), lambda i: (0, i)),
        ],
        out_specs=pl.BlockSpec((BT, H), lambda i: (i, 0)),
        out_shape=jax.ShapeDtypeStruct((NP, H), jnp.float32),
    )(xp, w1, deg2)


def _tc_mid_body(a_ref, d_ref, b_ref, w_ref, o_ref):
    dis = _dis_block(d_ref)
    h = (a_ref[0] + a_ref[1]) * dis[:, None] + b_ref[0, :][None, :]
    act = jnp.maximum(h, 0.0)
    o_ref[...] = jnp.dot(act, w_ref[...],
                         preferred_element_type=jnp.float32) * dis[:, None]


def _tc_mid(acc, deg2, b_prev, w_next):
    return pl.pallas_call(
        _tc_mid_body,
        grid=(GRID,),
        in_specs=[
            pl.BlockSpec((NC, BT, H), lambda i: (0, i, 0)),
            pl.BlockSpec((NC, BT), lambda i: (0, i)),
            pl.BlockSpec((1, H), lambda i: (0, 0)),
            pl.BlockSpec((H, H), lambda i: (0, 0)),
        ],
        out_specs=pl.BlockSpec((BT, H), lambda i: (i, 0)),
        out_shape=jax.ShapeDtypeStruct((NP, H), jnp.float32),
    )(acc, deg2, b_prev, w_next)


def _tc_head_body(a_ref, d_ref, b_ref, wh_ref, bh_ref, s_ref, w_ref):
    dis = _dis_block(d_ref)
    h = (a_ref[0] + a_ref[1]) * dis[:, None] + b_ref[0, :][None, :]
    act = jnp.maximum(h, 0.0)
    o = jnp.dot(act, wh_ref[...],
                preferred_element_type=jnp.float32) + bh_ref[0, :][None, :]
    s_ref[...] = o[:, 0:2]
    w_ref[...] = jax.nn.softplus(o[:, 2:3])


def _tc_head(acc, deg2, b3, wh, bh):
    return pl.pallas_call(
        _tc_head_body,
        grid=(GRID,),
        in_specs=[
            pl.BlockSpec((NC, BT, H), lambda i: (0, i, 0)),
            pl.BlockSpec((NC, BT), lambda i: (0, i)),
            pl.BlockSpec((1, H), lambda i: (0, 0)),
            pl.BlockSpec((H, 3), lambda i: (0, 0)),
            pl.BlockSpec((1, 3), lambda i: (0, 0)),
        ],
        out_specs=[
            pl.BlockSpec((BT, 2), lambda i: (i, 0)),
            pl.BlockSpec((BT, 1), lambda i: (i, 0)),
        ],
        out_shape=[
            jax.ShapeDtypeStruct((NP, 2), jnp.float32),
            jax.ShapeDtypeStruct((NP, 1), jnp.float32),
        ],
    )(acc, deg2, b3, wh, bh)


def kernel(x, edge_index, W1, b1, W2, b2, W3, b3, Ws, bs, Ww, bw):
    xp = jnp.zeros((NP, IN_C), jnp.float32).at[:N].set(x)
    ei = edge_index.astype(jnp.int32)
    pad_row = (N + (jnp.arange(G, dtype=jnp.int32) % (NP - N)))[:, None]
    srcs = jnp.concatenate(
        [ei[0].reshape(G, KR),
         jnp.broadcast_to(pad_row, (G, K - KR))], axis=1)
    dsts = jnp.concatenate(
        [ei[1].reshape(G, KR),
         jnp.broadcast_to(pad_row, (G, K - KR))], axis=1)
    ei_packed = jnp.stack([srcs, dsts], axis=1)
    dst_flat = ei[1]
    zeros_nph = jnp.zeros((NP, H), jnp.float32)
    wh = jnp.concatenate([Ws, Ww], axis=1)
    bh = jnp.concatenate([bs, bw]).reshape(1, 3)
    b1r = b1.reshape(1, H)
    b2r = b2.reshape(1, H)
    b3r = b3.reshape(1, H)

    deg2 = _deg_kernel(dst_flat)
    p1 = _tc_first(xp, W1, deg2)
    acc1 = _edge_kernel(p1, ei_packed, zeros_nph)
    p2 = _tc_mid(acc1, deg2, b1r, W2)
    acc2 = _edge_kernel(p2, ei_packed, zeros_nph)
    p3 = _tc_mid(acc2, deg2, b2r, W3)
    acc3 = _edge_kernel(p3, ei_packed, zeros_nph)
    shifts, weights = _tc_head(acc3, deg2, b3r, wh, bh)
    return (shifts[:N], weights[:N, 0])
